# baseline (speedup 1.0000x reference)
import functools

import jax
import jax.numpy as jnp
from jax import lax
from jax.experimental import pallas as pl
from jax.experimental.pallas import tpu as pltpu
from jax.experimental.pallas import tpu_sc as plsc

_K = 16
_NB = 8
_MAX_RADIUS = 10.0
_MUL_OUT = 32
_DIM_OUT = _MUL_OUT * 3
_ROWS = 512


def _insert_sorted(P, A, v, a):
    depth = len(P)
    cs = [v < Pd for Pd in P]
    newP, newA = [None] * depth, [None] * depth
    for d in range(depth - 1, 0, -1):
        newP[d] = jnp.where(cs[d - 1], P[d - 1], jnp.where(cs[d], v, P[d]))
        newA[d] = jnp.where(cs[d - 1], A[d - 1], jnp.where(cs[d], a, A[d]))
    newP[0] = jnp.where(cs[0], v, P[0])
    newA[0] = jnp.where(cs[0], a, A[0])
    return newP, newA


def _fold_topd(v0, a0, nchunk, depth):
    rows, width = v0.shape
    w = width // nchunk
    P = [jnp.full((rows, w), jnp.inf, jnp.float32) for _ in range(depth)]
    A = [jnp.zeros((rows, w), jnp.int32) for _ in range(depth)]
    for j in range(nchunk):
        P, A = _insert_sorted(P, A, v0[:, j * w:(j + 1) * w],
                              a0[:, j * w:(j + 1) * w])
    return jnp.concatenate(P, axis=1), jnp.concatenate(A, axis=1)


def _knn_geom(np_):
    if np_ % 10240 == 0:
        stages = [(16, 3), (5, 3), (9, 4)]
    else:
        stages = [(16, 3), (3, 3)]
    cand = np_
    for nc_, dp_ in stages:
        cand = dp_ * (cand // nc_)
    return stages, cand


def _knn_body(ct_ref, q_ref, idx_ref, d2_ref, p_ref, a_ref, *, n_valid, rows,
              stages):
    i = pl.program_id(0)
    q = q_ref[...]
    ct = ct_ref[...]
    d2 = None
    for m in range(3):
        diff = ct[m:m + 1, :] - q[:, m:m + 1]
        sq = diff * diff
        d2 = sq if d2 is None else d2 + sq
    col = lax.broadcasted_iota(jnp.int32, d2.shape, 1)
    d2_ref[...] = d2
    P2, A2 = d2, col
    for nc_, dp_ in stages:
        P2, A2 = _fold_topd(P2, A2, nc_, dp_)
    p_ref[...] = P2
    a_ref[...] = A2
    cols = []
    m = None
    for t in range(_K + 1):
        pv = p_ref[...]
        av = a_ref[...]
        m = jnp.min(pv, axis=1, keepdims=True)
        it = jnp.min(jnp.where(pv == m, av, jnp.int32(2 ** 30)),
                     axis=1, keepdims=True)
        cols.append(it)
        if t + 1 < _K + 1:
            p_ref[...] = jnp.where((pv == m) & (av == it), jnp.inf, pv)
    idx_ref[...] = jnp.concatenate(cols[1:], axis=1)
    d2v = d2_ref[...]
    cnt = jnp.sum(jnp.where(d2v <= m, 1.0, 0.0), axis=1, keepdims=True)
    rowg1 = i * rows + lax.broadcasted_iota(jnp.int32, (rows, 1), 0)
    cnt = jnp.where(rowg1 < n_valid, cnt, 0.0)
    bad = jnp.max(cnt) > float(_K + 1)

    @pl.when(bad)
    def _fallback():
        fcols = []
        fit = None
        for t in range(_K + 1):
            dv = d2_ref[...]
            if t > 0:
                dv = jnp.where(col == fit, jnp.inf, dv)
                d2_ref[...] = dv
            fm = jnp.min(dv, axis=1, keepdims=True)
            fit = jnp.min(jnp.where(dv == fm, col, jnp.int32(2 ** 30)),
                          axis=1, keepdims=True)
            fcols.append(fit)
        idx_ref[...] = jnp.concatenate(fcols[1:], axis=1)


def _build_knn(np_, n_valid, rows):
    stages, cand = _knn_geom(np_)
    return pl.pallas_call(
        functools.partial(_knn_body, n_valid=n_valid, rows=rows,
                          stages=stages),
        grid=(np_ // rows,),
        in_specs=[
            pl.BlockSpec((8, np_), lambda i: (0, 0)),
            pl.BlockSpec((rows, 8), lambda i: (i, 0)),
        ],
        out_specs=pl.BlockSpec((rows, _K), lambda i: (i, 0)),
        out_shape=jax.ShapeDtypeStruct((np_, _K), jnp.int32),
        scratch_shapes=[pltpu.VMEM((rows, np_), jnp.float32),
                        pltpu.VMEM((rows, cand), jnp.float32),
                        pltpu.VMEM((rows, cand), jnp.int32)],
    )


def _rsqrt_newton(x):
    bits = plsc.bitcast(x, jnp.int32)
    y = plsc.bitcast(jnp.int32(0x5F3759DF) - (bits >> 1), jnp.float32)
    for _ in range(3):
        y = y * (1.5 - 0.5 * x * y * y)
    return y


def _build_sc_feature(np_, nc, ns, lanes):
    nw = nc * ns
    bpw = np_ // nw
    ngrp = bpw // lanes
    scale = 1.0 / (_K * float(_NB) ** 0.5)
    mesh = plsc.VectorSubcoreMesh(core_axis_name="c", subcore_axis_name="s")

    @functools.partial(
        pl.kernel,
        mesh=mesh,
        compiler_params=pltpu.CompilerParams(needs_layout_passes=False),
        out_type=jax.ShapeDtypeStruct((np_ * _DIM_OUT,), jnp.float32),
        scratch_types=[
            pltpu.VMEM((np_,), jnp.float32),
            pltpu.VMEM((np_,), jnp.float32),
            pltpu.VMEM((np_,), jnp.float32),
            pltpu.VMEM((bpw * _K,), jnp.int32),
            pltpu.VMEM((_NB * _MUL_OUT * lanes,), jnp.float32),
            pltpu.VMEM((bpw * _DIM_OUT,), jnp.float32),
        ],
    )
    def k(x_hbm, y_hbm, z_hbm, idx_hbm, w_hbm, out_hbm,
          xv, yv, zv, idxv, wv, outv):
        wid = lax.axis_index("s") * nc + lax.axis_index("c")
        base = wid * bpw
        pltpu.sync_copy(x_hbm, xv)
        pltpu.sync_copy(y_hbm, yv)
        pltpu.sync_copy(z_hbm, zv)
        pltpu.sync_copy(idx_hbm.at[pl.ds(base * _K, bpw * _K)], idxv)
        pltpu.sync_copy(w_hbm, wv)

        lane = lax.iota(jnp.int32, lanes)

        def group(g, carry):
            gb = g * lanes
            xo = xv[pl.ds(base + gb, lanes)]
            yo = yv[pl.ds(base + gb, lanes)]
            zo = zv[pl.ds(base + gb, lanes)]

            def kstep(kk, S):
                nbr = plsc.load_gather(idxv, [lane * _K + (gb * _K + kk)])
                xn = plsc.load_gather(xv, [nbr])
                yn = plsc.load_gather(yv, [nbr])
                zn = plsc.load_gather(zv, [nbr])
                rx = xn - xo
                ry = yn - yo
                rz = zn - zo
                d2 = rx * rx + ry * ry + rz * rz
                rinv = _rsqrt_newton(d2 + 1e-30)
                d = d2 * rinv
                rdiv = 1.0 / (d + 1e-8)
                rnx = rx * rdiv
                rny = ry * rdiv
                rnz = rz * rdiv
                cut = jnp.minimum(d * (1.0 / _MAX_RADIUS), 1.0)
                es = []
                s = None
                for v in range(_NB):
                    c_v = v / (_NB - 1.0)
                    e = jnp.exp((cut - c_v) * (cut - c_v) * (-0.5 * _NB * _NB))
                    es.append(e)
                    s = e if s is None else s + e
                rs = 1.0 / s
                out = []
                for v in range(_NB):
                    t = es[v] * rs
                    out.append(S[3 * v] + t * rnx)
                    out.append(S[3 * v + 1] + t * rny)
                    out.append(S[3 * v + 2] + t * rnz)
                return tuple(out)

            zero = jnp.zeros((lanes,), jnp.float32)
            S = lax.fori_loop(0, _K, kstep, tuple([zero] * (3 * _NB)))

            for w in range(_MUL_OUT):
                a0 = zero
                a1 = zero
                a2 = zero
                for v in range(_NB):
                    wvw = wv[pl.ds((v * _MUL_OUT + w) * lanes, lanes)]
                    a0 = a0 + wvw * S[3 * v]
                    a1 = a1 + wvw * S[3 * v + 1]
                    a2 = a2 + wvw * S[3 * v + 2]
                obase = gb * _DIM_OUT + 3 * w
                plsc.store_scatter(outv, [lane * _DIM_OUT + obase], a0 * scale)
                plsc.store_scatter(outv, [lane * _DIM_OUT + (obase + 1)],
                                   a1 * scale)
                plsc.store_scatter(outv, [lane * _DIM_OUT + (obase + 2)],
                                   a2 * scale)
            return carry

        lax.fori_loop(0, ngrp, group, 0)
        pltpu.sync_copy(outv, out_hbm.at[pl.ds(base * _DIM_OUT,
                                               bpw * _DIM_OUT)])

    return k


def kernel(coords, tp_weight):
    b, n, _ = coords.shape
    np_ = -(-n // 512) * 512
    cf = coords.reshape(b * n, 3).astype(jnp.float32)
    cpad = jnp.full((np_, 8), 1e17, jnp.float32).at[:n, :3].set(cf)
    idx = _build_knn(np_, n, _ROWS)(cpad.T, cpad)

    info = plsc.get_sparse_core_info()
    feat = _build_sc_feature(np_, info.num_cores, info.num_subcores,
                             info.num_lanes)
    w_splat = jnp.repeat(tp_weight.reshape(-1), info.num_lanes)
    out_flat = feat(cpad[:, 0], cpad[:, 1], cpad[:, 2],
                    idx.reshape(-1), w_splat)
    return out_flat.reshape(np_, _DIM_OUT)[:n].reshape(b, n, _DIM_OUT)


# speedup vs baseline: 1.2794x; 1.2794x over previous
import functools

import jax
import jax.numpy as jnp
from jax import lax
from jax.experimental import pallas as pl
from jax.experimental.pallas import tpu as pltpu
from jax.experimental.pallas import tpu_sc as plsc

_K = 16
_NB = 8
_MAX_RADIUS = 10.0
_MUL_OUT = 32
_DIM_OUT = _MUL_OUT * 3
_ROWS = 256


def _insert_sorted(P, A, v, a):
    depth = len(P)
    cs = [v < Pd for Pd in P]
    newP, newA = [None] * depth, [None] * depth
    for d in range(depth - 1, 0, -1):
        newP[d] = jnp.where(cs[d - 1], P[d - 1], jnp.where(cs[d], v, P[d]))
        newA[d] = jnp.where(cs[d - 1], A[d - 1], jnp.where(cs[d], a, A[d]))
    newP[0] = jnp.where(cs[0], v, P[0])
    newA[0] = jnp.where(cs[0], a, A[0])
    return newP, newA


def _fold_topd(v0, a0, nchunk, depth):
    rows, width = v0.shape
    w = width // nchunk
    P = [jnp.full((rows, w), jnp.inf, jnp.float32) for _ in range(depth)]
    A = [jnp.zeros((rows, w), jnp.int32) for _ in range(depth)]
    for j in range(nchunk):
        P, A = _insert_sorted(P, A, v0[:, j * w:(j + 1) * w],
                              a0[:, j * w:(j + 1) * w])
    return jnp.concatenate(P, axis=1), jnp.concatenate(A, axis=1)


def _knn_geom(np_):
    if np_ % 10240 == 0:
        stages = [(16, 3), (5, 3), (9, 4)]
    else:
        stages = [(16, 3), (3, 3)]
    cand = np_
    for nc_, dp_ in stages:
        cand = dp_ * (cand // nc_)
    return stages, cand


def _knn_body(ct_ref, q_ref, idx_ref, d2_ref, p_ref, a_ref, *, n_valid, rows,
              stages, blk0=0):
    i = pl.program_id(0) + blk0
    q = q_ref[...]
    ct = ct_ref[...]
    d2 = None
    for m in range(3):
        diff = ct[m:m + 1, :] - q[:, m:m + 1]
        sq = diff * diff
        d2 = sq if d2 is None else d2 + sq
    col = lax.broadcasted_iota(jnp.int32, d2.shape, 1)
    d2_ref[...] = d2
    P2, A2 = d2, col
    for nc_, dp_ in stages:
        P2, A2 = _fold_topd(P2, A2, nc_, dp_)
    p_ref[...] = P2
    a_ref[...] = A2
    cols = []
    m = None
    for t in range(_K + 1):
        pv = p_ref[...]
        av = a_ref[...]
        m = jnp.min(pv, axis=1, keepdims=True)
        it = jnp.min(jnp.where(pv == m, av, jnp.int32(2 ** 30)),
                     axis=1, keepdims=True)
        cols.append(it)
        if t + 1 < _K + 1:
            p_ref[...] = jnp.where((pv == m) & (av == it), jnp.inf, pv)
    idx_ref[...] = jnp.concatenate(cols[1:], axis=1)
    d2v = d2_ref[...]
    cnt = jnp.sum(jnp.where(d2v <= m, 1.0, 0.0), axis=1, keepdims=True)
    rowg1 = i * rows + lax.broadcasted_iota(jnp.int32, (rows, 1), 0)
    cnt = jnp.where(rowg1 < n_valid, cnt, 0.0)
    bad = jnp.max(cnt) > float(_K + 1)

    @pl.when(bad)
    def _fallback():
        fcols = []
        fit = None
        for t in range(_K + 1):
            dv = d2_ref[...]
            if t > 0:
                dv = jnp.where(col == fit, jnp.inf, dv)
                d2_ref[...] = dv
            fm = jnp.min(dv, axis=1, keepdims=True)
            fit = jnp.min(jnp.where(dv == fm, col, jnp.int32(2 ** 30)),
                          axis=1, keepdims=True)
            fcols.append(fit)
        idx_ref[...] = jnp.concatenate(fcols[1:], axis=1)


def _build_knn(np_, n_valid, rows, row0=0, nrows=None):
    stages, cand = _knn_geom(np_)
    if nrows is None:
        nrows = np_
    blk0 = row0 // rows
    return pl.pallas_call(
        functools.partial(_knn_body, n_valid=n_valid, rows=rows,
                          stages=stages, blk0=blk0),
        grid=(nrows // rows,),
        in_specs=[
            pl.BlockSpec((8, np_), lambda i: (0, 0)),
            pl.BlockSpec((rows, 8), lambda i: (i + blk0, 0)),
        ],
        out_specs=pl.BlockSpec((rows, _K), lambda i: (i, 0)),
        out_shape=jax.ShapeDtypeStruct((nrows, _K), jnp.int32),
        scratch_shapes=[pltpu.VMEM((rows, np_), jnp.float32),
                        pltpu.VMEM((rows, cand), jnp.float32),
                        pltpu.VMEM((rows, cand), jnp.int32)],
    )


def _rsqrt_newton(x):
    bits = plsc.bitcast(x, jnp.int32)
    y = plsc.bitcast(jnp.int32(0x5F3759DF) - (bits >> 1), jnp.float32)
    for _ in range(3):
        y = y * (1.5 - 0.5 * x * y * y)
    return y


def _build_sc_feature(np_, nc, ns, lanes, node0=0, nnodes=None):
    if nnodes is None:
        nnodes = np_
    nw = nc * ns
    bpw = nnodes // nw
    ngrp = bpw // lanes
    scale = 1.0 / (_K * float(_NB) ** 0.5)
    mesh = plsc.VectorSubcoreMesh(core_axis_name="c", subcore_axis_name="s")

    @functools.partial(
        pl.kernel,
        mesh=mesh,
        compiler_params=pltpu.CompilerParams(needs_layout_passes=False),
        out_type=jax.ShapeDtypeStruct((nnodes * _DIM_OUT,), jnp.float32),
        scratch_types=[
            pltpu.VMEM((np_,), jnp.float32),
            pltpu.VMEM((np_,), jnp.float32),
            pltpu.VMEM((np_,), jnp.float32),
            pltpu.VMEM((bpw * _K,), jnp.int32),
            pltpu.VMEM((_NB * _MUL_OUT * lanes,), jnp.float32),
            pltpu.VMEM((bpw * _DIM_OUT,), jnp.float32),
        ],
    )
    def k(x_hbm, y_hbm, z_hbm, idx_hbm, w_hbm, out_hbm,
          xv, yv, zv, idxv, wv, outv):
        wid = lax.axis_index("s") * nc + lax.axis_index("c")
        base = wid * bpw
        pltpu.sync_copy(x_hbm, xv)
        pltpu.sync_copy(y_hbm, yv)
        pltpu.sync_copy(z_hbm, zv)
        pltpu.sync_copy(idx_hbm.at[pl.ds(base * _K, bpw * _K)], idxv)
        pltpu.sync_copy(w_hbm, wv)

        lane = lax.iota(jnp.int32, lanes)

        def group(g, carry):
            gb = g * lanes
            xo = xv[pl.ds(node0 + base + gb, lanes)]
            yo = yv[pl.ds(node0 + base + gb, lanes)]
            zo = zv[pl.ds(node0 + base + gb, lanes)]

            def kstep(kk, S):
                nbr = plsc.load_gather(idxv, [lane * _K + (gb * _K + kk)])
                xn = plsc.load_gather(xv, [nbr])
                yn = plsc.load_gather(yv, [nbr])
                zn = plsc.load_gather(zv, [nbr])
                rx = xn - xo
                ry = yn - yo
                rz = zn - zo
                d2 = rx * rx + ry * ry + rz * rz
                rinv = _rsqrt_newton(d2 + 1e-30)
                d = d2 * rinv
                rdiv = 1.0 / (d + 1e-8)
                rnx = rx * rdiv
                rny = ry * rdiv
                rnz = rz * rdiv
                cut = jnp.minimum(d * (1.0 / _MAX_RADIUS), 1.0)
                es = []
                s = None
                for v in range(_NB):
                    c_v = v / (_NB - 1.0)
                    e = jnp.exp((cut - c_v) * (cut - c_v) * (-0.5 * _NB * _NB))
                    es.append(e)
                    s = e if s is None else s + e
                rs = 1.0 / s
                out = []
                for v in range(_NB):
                    t = es[v] * rs
                    out.append(S[3 * v] + t * rnx)
                    out.append(S[3 * v + 1] + t * rny)
                    out.append(S[3 * v + 2] + t * rnz)
                return tuple(out)

            zero = jnp.zeros((lanes,), jnp.float32)
            S = lax.fori_loop(0, _K, kstep, tuple([zero] * (3 * _NB)))

            for w in range(_MUL_OUT):
                a0 = zero
                a1 = zero
                a2 = zero
                for v in range(_NB):
                    wvw = wv[pl.ds((v * _MUL_OUT + w) * lanes, lanes)]
                    a0 = a0 + wvw * S[3 * v]
                    a1 = a1 + wvw * S[3 * v + 1]
                    a2 = a2 + wvw * S[3 * v + 2]
                obase = gb * _DIM_OUT + 3 * w
                plsc.store_scatter(outv, [lane * _DIM_OUT + obase], a0 * scale)
                plsc.store_scatter(outv, [lane * _DIM_OUT + (obase + 1)],
                                   a1 * scale)
                plsc.store_scatter(outv, [lane * _DIM_OUT + (obase + 2)],
                                   a2 * scale)
            return carry

        lax.fori_loop(0, ngrp, group, 0)
        pltpu.sync_copy(outv, out_hbm.at[pl.ds(base * _DIM_OUT,
                                               bpw * _DIM_OUT)])

    return k


def kernel(coords, tp_weight):
    b, n, _ = coords.shape
    np_ = -(-n // 512) * 512
    cf = coords.reshape(b * n, 3).astype(jnp.float32)
    cpad = jnp.full((np_, 8), 1e17, jnp.float32).at[:n, :3].set(cf)
    info = plsc.get_sparse_core_info()
    w_splat = jnp.repeat(tp_weight.reshape(-1), info.num_lanes)
    ct = cpad.T
    x, y, z = cpad[:, 0], cpad[:, 1], cpad[:, 2]
    half = np_ // 2
    outs = []
    for node0 in (0, half):
        idx_h = _build_knn(np_, n, _ROWS, row0=node0, nrows=half)(ct, cpad)
        feat = _build_sc_feature(np_, info.num_cores, info.num_subcores,
                                 info.num_lanes, node0=node0, nnodes=half)
        outs.append(feat(x, y, z, idx_h.reshape(-1), w_splat))
    out_flat = jnp.concatenate(outs)
    return out_flat.reshape(np_, _DIM_OUT)[:n].reshape(b, n, _DIM_OUT)


# speedup vs baseline: 1.2942x; 1.0116x over previous
import functools

import jax
import jax.numpy as jnp
from jax import lax
from jax.experimental import pallas as pl
from jax.experimental.pallas import tpu as pltpu
from jax.experimental.pallas import tpu_sc as plsc

_K = 16
_NB = 8
_MAX_RADIUS = 10.0
_MUL_OUT = 32
_DIM_OUT = _MUL_OUT * 3
_ROWS = 256


def _insert_sorted(P, A, v, a):
    depth = len(P)
    cs = [v < Pd for Pd in P]
    newP, newA = [None] * depth, [None] * depth
    for d in range(depth - 1, 0, -1):
        newP[d] = jnp.where(cs[d - 1], P[d - 1], jnp.where(cs[d], v, P[d]))
        newA[d] = jnp.where(cs[d - 1], A[d - 1], jnp.where(cs[d], a, A[d]))
    newP[0] = jnp.where(cs[0], v, P[0])
    newA[0] = jnp.where(cs[0], a, A[0])
    return newP, newA


def _fold_topd(v0, a0, nchunk, depth):
    rows, width = v0.shape
    w = width // nchunk
    P = [jnp.full((rows, w), jnp.inf, jnp.float32) for _ in range(depth)]
    A = [jnp.zeros((rows, w), jnp.int32) for _ in range(depth)]
    for j in range(nchunk):
        P, A = _insert_sorted(P, A, v0[:, j * w:(j + 1) * w],
                              a0[:, j * w:(j + 1) * w])
    return jnp.concatenate(P, axis=1), jnp.concatenate(A, axis=1)


def _knn_geom(np_):
    if np_ % 10240 == 0:
        stages = [(16, 3), (5, 3), (9, 4)]
    else:
        stages = [(16, 3), (3, 3)]
    cand = np_
    for nc_, dp_ in stages:
        cand = dp_ * (cand // nc_)
    return stages, cand


def _knn_body(ct_ref, q_ref, idx_ref, d2_ref, p_ref, a_ref, *, n_valid, rows,
              stages):
    i = pl.program_id(0)
    q = q_ref[...]
    ct = ct_ref[...]
    d2 = None
    for m in range(3):
        diff = ct[m:m + 1, :] - q[:, m:m + 1]
        sq = diff * diff
        d2 = sq if d2 is None else d2 + sq
    col = lax.broadcasted_iota(jnp.int32, d2.shape, 1)
    d2_ref[...] = d2
    P2, A2 = d2, col
    for nc_, dp_ in stages:
        P2, A2 = _fold_topd(P2, A2, nc_, dp_)
    p_ref[...] = P2
    a_ref[...] = A2
    def _extract(pv, av):
        mm = jnp.min(pv, axis=1, keepdims=True)
        ii = jnp.min(jnp.where(pv == mm, av, jnp.int32(2 ** 30)),
                     axis=1, keepdims=True)
        return mm, ii

    def _mask(pv, av, mm, ii):
        return jnp.where((pv == mm) & (av == ii), jnp.inf, pv)

    cols = []
    m = None
    nex = _K + 1
    t = 0
    while t < nex:
        pv = p_ref[...]
        av = a_ref[...]
        m, it = _extract(pv, av)
        cols.append(it)
        t += 1
        if t < nex:
            pv = _mask(pv, av, m, it)
            m, it = _extract(pv, av)
            cols.append(it)
            t += 1
            if t < nex:
                p_ref[...] = _mask(pv, av, m, it)
    idx_ref[...] = jnp.concatenate(cols[1:], axis=1)
    d2v = d2_ref[...]
    cnt = jnp.sum(jnp.where(d2v <= m, 1.0, 0.0), axis=1, keepdims=True)
    rowg1 = i * rows + lax.broadcasted_iota(jnp.int32, (rows, 1), 0)
    cnt = jnp.where(rowg1 < n_valid, cnt, 0.0)
    bad = jnp.max(cnt) > float(_K + 1)

    @pl.when(bad)
    def _fallback():
        fcols = []
        fit = None
        for t in range(_K + 1):
            dv = d2_ref[...]
            if t > 0:
                dv = jnp.where(col == fit, jnp.inf, dv)
                d2_ref[...] = dv
            fm = jnp.min(dv, axis=1, keepdims=True)
            fit = jnp.min(jnp.where(dv == fm, col, jnp.int32(2 ** 30)),
                          axis=1, keepdims=True)
            fcols.append(fit)
        idx_ref[...] = jnp.concatenate(fcols[1:], axis=1)


def _build_knn(np_, n_valid, rows):
    stages, cand = _knn_geom(np_)
    return pl.pallas_call(
        functools.partial(_knn_body, n_valid=n_valid, rows=rows,
                          stages=stages),
        grid=(np_ // rows,),
        in_specs=[
            pl.BlockSpec((8, np_), lambda i: (0, 0)),
            pl.BlockSpec((rows, 8), lambda i: (i, 0)),
        ],
        out_specs=pl.BlockSpec((rows, _K), lambda i: (i, 0)),
        out_shape=jax.ShapeDtypeStruct((np_, _K), jnp.int32),
        scratch_shapes=[pltpu.VMEM((rows, np_), jnp.float32),
                        pltpu.VMEM((rows, cand), jnp.float32),
                        pltpu.VMEM((rows, cand), jnp.int32)],
    )


def _rsqrt_newton(x):
    bits = plsc.bitcast(x, jnp.int32)
    y = plsc.bitcast(jnp.int32(0x5F3759DF) - (bits >> 1), jnp.float32)
    for _ in range(3):
        y = y * (1.5 - 0.5 * x * y * y)
    return y


def _build_sc_feature(np_, nc, ns, lanes):
    nw = nc * ns
    bpw = np_ // nw
    ngrp = bpw // lanes
    scale = 1.0 / (_K * float(_NB) ** 0.5)
    mesh = plsc.VectorSubcoreMesh(core_axis_name="c", subcore_axis_name="s")

    @functools.partial(
        pl.kernel,
        mesh=mesh,
        compiler_params=pltpu.CompilerParams(needs_layout_passes=False),
        out_type=jax.ShapeDtypeStruct((np_ * _DIM_OUT,), jnp.float32),
        scratch_types=[
            pltpu.VMEM((np_,), jnp.float32),
            pltpu.VMEM((np_,), jnp.float32),
            pltpu.VMEM((np_,), jnp.float32),
            pltpu.VMEM((bpw * _K,), jnp.int32),
            pltpu.VMEM((_NB * _MUL_OUT * lanes,), jnp.float32),
            pltpu.VMEM((bpw * _DIM_OUT,), jnp.float32),
        ],
    )
    def k(x_hbm, y_hbm, z_hbm, idx_hbm, w_hbm, out_hbm,
          xv, yv, zv, idxv, wv, outv):
        wid = lax.axis_index("s") * nc + lax.axis_index("c")
        base = wid * bpw
        pltpu.sync_copy(x_hbm, xv)
        pltpu.sync_copy(y_hbm, yv)
        pltpu.sync_copy(z_hbm, zv)
        pltpu.sync_copy(idx_hbm.at[pl.ds(base * _K, bpw * _K)], idxv)
        pltpu.sync_copy(w_hbm, wv)

        lane = lax.iota(jnp.int32, lanes)

        def group(g, carry):
            gb = g * lanes
            xo = xv[pl.ds(base + gb, lanes)]
            yo = yv[pl.ds(base + gb, lanes)]
            zo = zv[pl.ds(base + gb, lanes)]

            def kstep(kk, S):
                nbr = plsc.load_gather(idxv, [lane * _K + (gb * _K + kk)])
                xn = plsc.load_gather(xv, [nbr])
                yn = plsc.load_gather(yv, [nbr])
                zn = plsc.load_gather(zv, [nbr])
                rx = xn - xo
                ry = yn - yo
                rz = zn - zo
                d2 = rx * rx + ry * ry + rz * rz
                rinv = _rsqrt_newton(d2 + 1e-30)
                d = d2 * rinv
                rdiv = 1.0 / (d + 1e-8)
                rnx = rx * rdiv
                rny = ry * rdiv
                rnz = rz * rdiv
                cut = jnp.minimum(d * (1.0 / _MAX_RADIUS), 1.0)
                es = []
                s = None
                for v in range(_NB):
                    c_v = v / (_NB - 1.0)
                    e = jnp.exp((cut - c_v) * (cut - c_v) * (-0.5 * _NB * _NB))
                    es.append(e)
                    s = e if s is None else s + e
                rs = 1.0 / s
                out = []
                for v in range(_NB):
                    t = es[v] * rs
                    out.append(S[3 * v] + t * rnx)
                    out.append(S[3 * v + 1] + t * rny)
                    out.append(S[3 * v + 2] + t * rnz)
                return tuple(out)

            zero = jnp.zeros((lanes,), jnp.float32)
            S = lax.fori_loop(0, _K, kstep, tuple([zero] * (3 * _NB)))

            for w in range(_MUL_OUT):
                a0 = zero
                a1 = zero
                a2 = zero
                for v in range(_NB):
                    wvw = wv[pl.ds((v * _MUL_OUT + w) * lanes, lanes)]
                    a0 = a0 + wvw * S[3 * v]
                    a1 = a1 + wvw * S[3 * v + 1]
                    a2 = a2 + wvw * S[3 * v + 2]
                obase = gb * _DIM_OUT + 3 * w
                plsc.store_scatter(outv, [lane * _DIM_OUT + obase], a0 * scale)
                plsc.store_scatter(outv, [lane * _DIM_OUT + (obase + 1)],
                                   a1 * scale)
                plsc.store_scatter(outv, [lane * _DIM_OUT + (obase + 2)],
                                   a2 * scale)
            return carry

        lax.fori_loop(0, ngrp, group, 0)
        pltpu.sync_copy(outv, out_hbm.at[pl.ds(base * _DIM_OUT,
                                               bpw * _DIM_OUT)])

    return k


def kernel(coords, tp_weight):
    b, n, _ = coords.shape
    np_ = -(-n // 512) * 512
    cf = coords.reshape(b * n, 3).astype(jnp.float32)
    cpad = jnp.full((np_, 8), 1e17, jnp.float32).at[:n, :3].set(cf)
    idx = _build_knn(np_, n, _ROWS)(cpad.T, cpad)

    info = plsc.get_sparse_core_info()
    feat = _build_sc_feature(np_, info.num_cores, info.num_subcores,
                             info.num_lanes)
    w_splat = jnp.repeat(tp_weight.reshape(-1), info.num_lanes)
    out_flat = feat(cpad[:, 0], cpad[:, 1], cpad[:, 2],
                    idx.reshape(-1), w_splat)
    return out_flat.reshape(np_, _DIM_OUT)[:n].reshape(b, n, _DIM_OUT)
